# Initial kernel scaffold; baseline (speedup 1.0000x reference)
import jax
import jax.numpy as jnp
from jax.experimental import pallas as pl


def kernel(tok_batch, tok_lens, table):
    raise NotImplementedError("write your pallas kernel here")


# speedup vs baseline: 6.5619x; 6.5619x over previous
import functools

import jax
import jax.numpy as jnp
from jax import lax
from jax.experimental import pallas as pl
from jax.experimental.pallas import tpu as pltpu
from jax.experimental.pallas import tpu_sc as plsc

NC = 2
NS = 16
NW = NC * NS
B = 4096
L = 50
D = 64
EPW = B // NW
OPW = EPW // 2
ND = D // 16

_mesh = plsc.VectorSubcoreMesh(
    core_axis_name="c", subcore_axis_name="s", num_cores=NC, num_subcores=NS)


@functools.partial(
    pl.kernel,
    out_type=jax.ShapeDtypeStruct((B, D), jnp.float32),
    mesh=_mesh,
    scratch_types=[
        pltpu.VMEM((OPW, 2 * L), jnp.int32),
        pltpu.VMEM((2 * L, D), jnp.float32),
        pltpu.VMEM((EPW, D), jnp.float32),
        pltpu.VMEM((EPW, 16), jnp.int32),
        pltpu.SemaphoreType.DMA,
    ],
    compiler_params=pltpu.CompilerParams(use_tc_tiling_on_sc=False),
)
def _encode(tok2, lens, table, out, idx_v, buf, outb, lens_v, sem):
    wid = lax.axis_index("c") * NS + lax.axis_index("s")
    base = wid * EPW
    pltpu.sync_copy(tok2.at[pl.ds(wid * OPW, OPW)], idx_v)
    pltpu.sync_copy(lens.at[pl.ds(base, EPW)], lens_v)

    @pl.loop(0, OPW)
    def _per_op(j):
        pltpu.async_copy(table.at[idx_v.at[j]], buf, sem).wait()
        for e in range(2):
            eloc = 2 * j + e
            accs = [buf[L * e, pl.ds(d * 16, 16)] for d in range(ND)]
            for r in range(1, L):
                for d in range(ND):
                    accs[d] = accs[d] + buf[L * e + r, pl.ds(d * 16, 16)]
            lvec = lens_v[eloc, pl.ds(0, 16)]
            inv = 1.0 / lvec.astype(jnp.float32)
            for d in range(ND):
                outb[eloc, pl.ds(d * 16, 16)] = accs[d] * inv

    pltpu.sync_copy(outb, out.at[pl.ds(base, EPW)])


def kernel(tok_batch, tok_lens, table):
    tok2 = tok_batch.reshape(B // 2, 2 * L)
    lens16 = jnp.broadcast_to(tok_lens[:, None], (B, 16))
    return _encode(tok2, lens16, table)


# speedup vs baseline: 7.9284x; 1.2083x over previous
import functools

import jax
import jax.numpy as jnp
from jax import lax
from jax.experimental import pallas as pl
from jax.experimental.pallas import tpu as pltpu
from jax.experimental.pallas import tpu_sc as plsc

NC = 2
NS = 16
NW = NC * NS
B = 4096
L = 50
D = 64
EPW = B // NW
OPW = EPW // 2
ND = D // 16
NBUF = 4

_mesh = plsc.VectorSubcoreMesh(
    core_axis_name="c", subcore_axis_name="s", num_cores=NC, num_subcores=NS)


@functools.partial(
    pl.kernel,
    out_type=jax.ShapeDtypeStruct((B, D), jnp.float32),
    mesh=_mesh,
    scratch_types=[
        pltpu.VMEM((OPW, 2 * L), jnp.int32),
        [pltpu.VMEM((2 * L, D), jnp.float32) for _ in range(NBUF)],
        pltpu.VMEM((EPW, D), jnp.float32),
        pltpu.VMEM((EPW, 16), jnp.int32),
        [pltpu.SemaphoreType.DMA for _ in range(NBUF)],
    ],
    compiler_params=pltpu.CompilerParams(use_tc_tiling_on_sc=False),
)
def _encode(tok2, lens, table, out, idx_v, bufs, outb, lens_v, sems):
    wid = lax.axis_index("c") * NS + lax.axis_index("s")
    base = wid * EPW
    pltpu.sync_copy(tok2.at[pl.ds(wid * OPW, OPW)], idx_v)
    pltpu.sync_copy(lens.at[pl.ds(base, EPW)], lens_v)

    def start(j, b):
        pltpu.async_copy(table.at[idx_v.at[j]], bufs[b], sems[b])

    for b in range(NBUF):
        start(b, b)

    @pl.loop(0, OPW, step=NBUF)
    def _per_group(j0):
        for b in range(NBUF):
            j = j0 + b
            pltpu.make_async_copy(
                table.at[idx_v.at[j]], bufs[b], sems[b]).wait()
            buf = bufs[b]
            for e in range(2):
                eloc = 2 * j + e
                accs = [buf[L * e, pl.ds(d * 16, 16)] for d in range(ND)]
                for r in range(1, L):
                    for d in range(ND):
                        accs[d] = accs[d] + buf[L * e + r, pl.ds(d * 16, 16)]
                lvec = lens_v[eloc, pl.ds(0, 16)]
                inv = 1.0 / lvec.astype(jnp.float32)
                for d in range(ND):
                    outb[eloc, pl.ds(d * 16, 16)] = accs[d] * inv

            @pl.when(j + NBUF < OPW)
            def _refill():
                start(j + NBUF, b)

    pltpu.sync_copy(outb, out.at[pl.ds(base, EPW)])


def kernel(tok_batch, tok_lens, table):
    tok2 = tok_batch.reshape(B // 2, 2 * L)
    lens16 = jnp.broadcast_to(tok_lens[:, None], (B, 16))
    return _encode(tok2, lens16, table)
